# baseline (speedup 1.0000x reference)
import functools

import jax
import jax.numpy as jnp
from jax import lax
from jax.experimental import pallas as pl
from jax.experimental.pallas import tpu as pltpu
from jax.experimental.pallas import tpu_sc as plsc

N = 10000
E = 320000
H = 128
NLAYER = 5
NP = 10112

NC = 2
NS = 16
NW = NC * NS

EC = 80
E_PER = E // NW
NCH = E_PER // EC
NB = 22 * 6 * 2
NBG = NB // 8
NODE_PER = NP // NS

_mesh = plsc.VectorSubcoreMesh(core_axis_name="c", subcore_axis_name="s",
                               num_cores=NC, num_subcores=NS)


def _relu_add_rows(xrows_v, erows_v):
    def row_body(r4, carry):
        for u in range(4):
            r = r4 * 4 + u
            for jj in range(H // 16):
                sl = pl.ds(jj * 16, 16)
                v = xrows_v[r, sl] + erows_v[r, sl]
                xrows_v[r, sl] = jnp.maximum(v, 0.0)
        return carry
    lax.fori_loop(0, EC // 4, row_body, 0)


def _zero_buf(buf, rows):
    zeros = jnp.zeros((16,), jnp.float32)
    def row_body(r, carry):
        for jj in range(H // 16):
            buf[r, pl.ds(jj * 16, 16)] = zeros
        return carry
    lax.fori_loop(0, rows, row_body, 0)


@functools.partial(
    pl.kernel,
    out_type=(
        jax.ShapeDtypeStruct((NP, H), jnp.float32),
        jax.ShapeDtypeStruct((NB, H), jnp.float32),
        jax.ShapeDtypeStruct((E,), jnp.int32),
    ),
    mesh=_mesh,
    scratch_types=[
        pltpu.VMEM((EC,), jnp.int32),
        pltpu.VMEM((EC, H), jnp.float32),
        pltpu.VMEM((22, H), jnp.float32),
        pltpu.VMEM((6, H), jnp.float32),
        pltpu.VMEM((2, H), jnp.float32),
        [[pltpu.VMEM((EC,), jnp.int32)] * 3] * 2,
        [pltpu.VMEM((EC,), jnp.int32)] * 2,
        pltpu.SemaphoreType.DMA,
        [pltpu.SemaphoreType.DMA] * 2,
    ],
)
def _encode_kernel(atom_table, atom_idx, bond0, bond1, bond2,
                   ea0, ea1, ea2,
                   x0_out, etab_out, code_out,
                   idx_v, rows_v, b0_v, b1_v, b2_v,
                   ea_v, code_v, sem, sie):
    c = lax.axis_index("c")
    s = lax.axis_index("s")
    wid = s * NC + c

    n_chunks = N // EC
    for k in range((n_chunks + NW - 1) // NW):
        ch = wid + NW * k
        @pl.when(ch < n_chunks)
        def _():
            base = ch * EC
            pltpu.sync_copy(atom_idx.at[pl.ds(base, EC)], idx_v)
            pltpu.async_copy(atom_table.at[idx_v], rows_v, sem).wait()
            pltpu.sync_copy(rows_v, x0_out.at[pl.ds(base, EC)])

    @pl.when(wid == 0)
    def _():
        _zero_buf(rows_v, EC)
        pltpu.sync_copy(rows_v.at[pl.ds(0, EC)], x0_out.at[pl.ds(N, EC)])
        pltpu.sync_copy(rows_v.at[pl.ds(0, NP - N - EC)],
                        x0_out.at[pl.ds(N + EC, NP - N - EC)])

    pltpu.sync_copy(bond0, b0_v)
    pltpu.sync_copy(bond1, b1_v)
    pltpu.sync_copy(bond2, b2_v)
    for k in range((NBG + NW - 1) // NW):
        g = wid + NW * k
        @pl.when(g < NBG)
        def _():
            for j in range(8):
                r = g * 8 + j
                a = r // 12
                b = (r % 12) // 2
                cc = r % 2
                for jj in range(H // 16):
                    sl = pl.ds(jj * 16, 16)
                    rows_v[j, sl] = b0_v[a, sl] + b1_v[b, sl] + b2_v[cc, sl]
            pltpu.sync_copy(rows_v.at[pl.ds(0, 8)],
                            etab_out.at[pl.ds(g * 8, 8)])

    def load_cols(i, q):
        sl = pl.ds(wid * E_PER + i * EC, EC)
        return [pltpu.async_copy(ea0.at[sl], ea_v[q][0], sie[q]),
                pltpu.async_copy(ea1.at[sl], ea_v[q][1], sie[q]),
                pltpu.async_copy(ea2.at[sl], ea_v[q][2], sie[q])]

    def compute_codes(i, q, descs):
        for d in descs:
            d.wait()
        for jj in range(EC // 16):
            vsl = pl.ds(jj * 16, 16)
            code_v[q][vsl] = (ea_v[q][0][vsl] * 12 + ea_v[q][1][vsl] * 2
                             + ea_v[q][2][vsl])
        pltpu.sync_copy(code_v[q],
                        code_out.at[pl.ds(wid * E_PER + i * EC, EC)])

    def cpair_body(k, carry):
        dA = load_cols(2 * k, 0)
        dB = load_cols(2 * k + 1, 1)
        compute_codes(2 * k, 0, dA)
        compute_codes(2 * k + 1, 1, dB)
        return carry
    lax.fori_loop(0, NCH // 2, cpair_body, 0)
    compute_codes(NCH - 1, 0, load_cols(NCH - 1, 0))


@functools.partial(
    pl.kernel,
    out_type=jax.ShapeDtypeStruct((NC, NP, H), jnp.float32),
    mesh=_mesh,
    scratch_types=[
        pltpu.VMEM_SHARED((NB, H), jnp.float32),
        pltpu.VMEM_SHARED((NP, H), jnp.float32),
        [pltpu.VMEM((EC,), jnp.int32)] * 2,
        [pltpu.VMEM((EC,), jnp.int32)] * 2,
        [pltpu.VMEM((EC,), jnp.int32)] * 2,
        [pltpu.VMEM((EC, H), jnp.float32)] * 2,
        [pltpu.VMEM((EC, H), jnp.float32)] * 2,
        [[pltpu.SemaphoreType.DMA] * 2] * 2,
        pltpu.SemaphoreType.DMA,
        pltpu.SemaphoreType.DMA,
        pltpu.SemaphoreType.DMA,
    ],
)
def _layer_sc_kernel(x, src, dst, code, etab,
                     agg_out,
                     etab_sh, agg_sh, sr, dx, co, xr, er, sg, ssc, sia, sib):
    c = lax.axis_index("c")
    s = lax.axis_index("s")
    wid = s * NC + c

    @pl.when(s == 0)
    def _():
        pltpu.sync_copy(etab, etab_sh)

    _zero_buf(xr[0], EC)
    node_base = s * NODE_PER
    off = 0
    while off < NODE_PER:
        size = min(EC, NODE_PER - off)
        pltpu.sync_copy(xr[0].at[pl.ds(0, size)],
                        agg_sh.at[pl.ds(node_base + off, size)])
        off += size
    plsc.subcore_barrier()

    def load_idx_async(i, q, sem):
        sl = pl.ds(wid * E_PER + i * EC, EC)
        return [pltpu.async_copy(src.at[sl], sr[q], sem),
                pltpu.async_copy(code.at[sl], co[q], sem),
                pltpu.async_copy(dst.at[sl], dx[q], sem)]

    def pair_body(k, carry):
        idxA = load_idx_async(2 * k, 0, sia)
        idxB = load_idx_async(2 * k + 1, 1, sib)
        for d in idxA:
            d.wait()
        gxA = pltpu.async_copy(x.at[sr[0]], xr[0], sg[0][0])
        geA = pltpu.async_copy(etab_sh.at[co[0]], er[0], sg[0][1])
        for d in idxB:
            d.wait()
        gxB = pltpu.async_copy(x.at[sr[1]], xr[1], sg[1][0])
        geA.wait()
        geB = pltpu.async_copy(etab_sh.at[co[1]], er[1], sg[1][1])
        gxA.wait()
        _relu_add_rows(xr[0], er[0])
        gxB.wait()
        geB.wait()
        scA = pltpu.async_copy(xr[0], agg_sh.at[dx[0]], ssc, add=True)
        _relu_add_rows(xr[1], er[1])
        scA.wait()
        pltpu.sync_copy(xr[1], agg_sh.at[dx[1]], add=True)
        return carry
    lax.fori_loop(0, NCH // 2, pair_body, 0)

    for d in load_idx_async(NCH - 1, 0, sia):
        d.wait()
    gxA = pltpu.async_copy(x.at[sr[0]], xr[0], sg[0][0])
    geA = pltpu.async_copy(etab_sh.at[co[0]], er[0], sg[0][1])
    gxA.wait()
    geA.wait()
    _relu_add_rows(xr[0], er[0])
    pltpu.sync_copy(xr[0], agg_sh.at[dx[0]], add=True)
    plsc.subcore_barrier()

    off = 0
    while off < NODE_PER:
        size = min(EC, NODE_PER - off)
        pltpu.sync_copy(agg_sh.at[pl.ds(node_base + off, size)],
                        agg_out.at[c, pl.ds(node_base + off, size)])
        off += size


ROWS_BLK = NODE_PER
N_BLKS = NP // ROWS_BLK


def _tc_layer_body(x_ref, agg_ref, w_ref, b_ref, xn_ref, pooled_ref,
                   pooledx_ref):
    i = pl.program_id(0)
    xb = x_ref[...]
    acc = xb + agg_ref[0] + agg_ref[1]
    y = jnp.dot(acc, w_ref[...], preferred_element_type=jnp.float32)
    y = jnp.maximum(y + b_ref[...], 0.0)
    rows = i * ROWS_BLK + lax.broadcasted_iota(jnp.int32, (ROWS_BLK, 1), 0)
    valid = rows < N
    y = jnp.where(valid, y, 0.0)
    xn_ref[...] = y
    ps = jnp.sum(y, axis=0, keepdims=True)

    @pl.when(i == 0)
    def _():
        pooled_ref[...] = ps

    @pl.when(i > 0)
    def _():
        pooled_ref[...] += ps

    if pooledx_ref is not None:
        pxs = jnp.sum(jnp.where(valid, xb, 0.0), axis=0, keepdims=True)

        @pl.when(i == 0)
        def _():
            pooledx_ref[...] = pxs

        @pl.when(i > 0)
        def _():
            pooledx_ref[...] += pxs


def _make_tc_layer(with_x_pool):
    out_shapes = [
        jax.ShapeDtypeStruct((NP, H), jnp.float32),
        jax.ShapeDtypeStruct((1, H), jnp.float32),
    ]
    out_specs = [
        pl.BlockSpec((ROWS_BLK, H), lambda i: (i, 0)),
        pl.BlockSpec((1, H), lambda i: (0, 0)),
    ]
    if with_x_pool:
        out_shapes.append(jax.ShapeDtypeStruct((1, H), jnp.float32))
        out_specs.append(pl.BlockSpec((1, H), lambda i: (0, 0)))
        body = _tc_layer_body
    else:
        def body(x_ref, agg_ref, w_ref, b_ref, xn_ref, pooled_ref):
            _tc_layer_body(x_ref, agg_ref, w_ref, b_ref, xn_ref, pooled_ref,
                           None)
    return pl.pallas_call(
        body,
        grid=(N_BLKS,),
        in_specs=[
            pl.BlockSpec((ROWS_BLK, H), lambda i: (i, 0)),
            pl.BlockSpec((NC, ROWS_BLK, H), lambda i: (0, i, 0)),
            pl.BlockSpec((H, H), lambda i: (0, 0)),
            pl.BlockSpec((1, H), lambda i: (0, 0)),
        ],
        out_specs=out_specs,
        out_shape=out_shapes,
    )


_tc_layer_first = _make_tc_layer(True)
_tc_layer_rest = _make_tc_layer(False)


def _head_body(pooled_ref, linw_ref, linb_ref, out_ref):
    s = jnp.sum(pooled_ref[...] * linw_ref[...]) + linb_ref[0, 0]
    out_ref[...] = s.reshape(1, 1)


_head = pl.pallas_call(
    _head_body,
    out_shape=jax.ShapeDtypeStruct((1, 1), jnp.float32),
)


@jax.jit
def kernel(atom_table, bond0, bond1, bond2, W, b, lin_W, lin_b,
           atom_idx, edge_index, edge_attr):
    atom_idx = atom_idx.astype(jnp.int32)
    edge_index = edge_index.astype(jnp.int32)
    edge_attr = edge_attr.astype(jnp.int32)
    src = edge_index[0]
    dst = edge_index[1]
    ea0 = edge_attr[:, 0]
    ea1 = edge_attr[:, 1]
    ea2 = edge_attr[:, 2]

    x0, etab, code = _encode_kernel(atom_table, atom_idx, bond0, bond1,
                                    bond2, ea0, ea1, ea2)

    pooled = []
    x = x0
    for l in range(NLAYER):
        agg2 = _layer_sc_kernel(x, src, dst, code, etab)
        wl = W[l]
        bl = b[l].reshape(1, H)
        if l == 0:
            x, p, p0 = _tc_layer_first(x, agg2, wl, bl)
            pooled.append(p0)
        else:
            x, p = _tc_layer_rest(x, agg2, wl, bl)
        pooled.append(p)

    pooled_all = jnp.concatenate(pooled, axis=0)
    pooled_all = jnp.pad(pooled_all, ((0, 2), (0, 0)))
    linw = jnp.pad(lin_W.reshape(NLAYER + 1, H), ((0, 2), (0, 0)))
    linb = lin_b.reshape(1, 1)
    out = _head(pooled_all, linw, linb)
    return out.reshape(1)


# speedup vs baseline: 1.1136x; 1.1136x over previous
import functools

import jax
import jax.numpy as jnp
from jax import lax
from jax.experimental import pallas as pl
from jax.experimental.pallas import tpu as pltpu
from jax.experimental.pallas import tpu_sc as plsc

N = 10000
E = 320000
H = 128
NLAYER = 5
NP = 10112

NC = 2
NS = 16
NW = NC * NS

EC = 80
E_PER = E // NW
NCH = E_PER // EC
NB = 22 * 6 * 2
NBG = NB // 8
NODE_PER = NP // NS

_mesh = plsc.VectorSubcoreMesh(core_axis_name="c", subcore_axis_name="s",
                               num_cores=NC, num_subcores=NS)


def _relu_add_rows(xrows_v, erows_v):
    def row_body(r2, carry):
        for u in range(2):
            r = r2 * 2 + u
            for jj in range(H // 16):
                sl = pl.ds(jj * 16, 16)
                v = xrows_v[r, sl] + erows_v[r, sl]
                xrows_v[r, sl] = jnp.maximum(v, 0.0)
        return carry
    lax.fori_loop(0, EC // 2, row_body, 0)


def _zero_buf(buf, rows):
    zeros = jnp.zeros((16,), jnp.float32)
    def row_body(r, carry):
        for jj in range(H // 16):
            buf[r, pl.ds(jj * 16, 16)] = zeros
        return carry
    lax.fori_loop(0, rows, row_body, 0)


@functools.partial(
    pl.kernel,
    out_type=(
        jax.ShapeDtypeStruct((NP, H), jnp.float32),
        jax.ShapeDtypeStruct((NB, H), jnp.float32),
        jax.ShapeDtypeStruct((E,), jnp.int32),
    ),
    mesh=_mesh,
    scratch_types=[
        pltpu.VMEM((EC,), jnp.int32),
        pltpu.VMEM((EC, H), jnp.float32),
        pltpu.VMEM((22, H), jnp.float32),
        pltpu.VMEM((6, H), jnp.float32),
        pltpu.VMEM((2, H), jnp.float32),
        [[pltpu.VMEM((EC,), jnp.int32)] * 3] * 2,
        [pltpu.VMEM((EC,), jnp.int32)] * 2,
        pltpu.SemaphoreType.DMA,
        [pltpu.SemaphoreType.DMA] * 2,
    ],
)
def _encode_kernel(atom_table, atom_idx, bond0, bond1, bond2,
                   ea0, ea1, ea2,
                   x0_out, etab_out, code_out,
                   idx_v, rows_v, b0_v, b1_v, b2_v,
                   ea_v, code_v, sem, sie):
    c = lax.axis_index("c")
    s = lax.axis_index("s")
    wid = s * NC + c

    n_chunks = N // EC
    for k in range((n_chunks + NW - 1) // NW):
        ch = wid + NW * k
        @pl.when(ch < n_chunks)
        def _():
            base = ch * EC
            pltpu.sync_copy(atom_idx.at[pl.ds(base, EC)], idx_v)
            pltpu.async_copy(atom_table.at[idx_v], rows_v, sem).wait()
            pltpu.sync_copy(rows_v, x0_out.at[pl.ds(base, EC)])

    @pl.when(wid == 0)
    def _():
        _zero_buf(rows_v, EC)
        pltpu.sync_copy(rows_v.at[pl.ds(0, EC)], x0_out.at[pl.ds(N, EC)])
        pltpu.sync_copy(rows_v.at[pl.ds(0, NP - N - EC)],
                        x0_out.at[pl.ds(N + EC, NP - N - EC)])

    pltpu.sync_copy(bond0, b0_v)
    pltpu.sync_copy(bond1, b1_v)
    pltpu.sync_copy(bond2, b2_v)
    for k in range((NBG + NW - 1) // NW):
        g = wid + NW * k
        @pl.when(g < NBG)
        def _():
            for j in range(8):
                r = g * 8 + j
                a = r // 12
                b = (r % 12) // 2
                cc = r % 2
                for jj in range(H // 16):
                    sl = pl.ds(jj * 16, 16)
                    rows_v[j, sl] = b0_v[a, sl] + b1_v[b, sl] + b2_v[cc, sl]
            pltpu.sync_copy(rows_v.at[pl.ds(0, 8)],
                            etab_out.at[pl.ds(g * 8, 8)])

    def load_cols(i, q):
        sl = pl.ds(wid * E_PER + i * EC, EC)
        return [pltpu.async_copy(ea0.at[sl], ea_v[q][0], sie[q]),
                pltpu.async_copy(ea1.at[sl], ea_v[q][1], sie[q]),
                pltpu.async_copy(ea2.at[sl], ea_v[q][2], sie[q])]

    def compute_codes(i, q, descs):
        for d in descs:
            d.wait()
        for jj in range(EC // 16):
            vsl = pl.ds(jj * 16, 16)
            code_v[q][vsl] = (ea_v[q][0][vsl] * 12 + ea_v[q][1][vsl] * 2
                             + ea_v[q][2][vsl])
        pltpu.sync_copy(code_v[q],
                        code_out.at[pl.ds(wid * E_PER + i * EC, EC)])

    def cpair_body(k, carry):
        dA = load_cols(2 * k, 0)
        dB = load_cols(2 * k + 1, 1)
        compute_codes(2 * k, 0, dA)
        compute_codes(2 * k + 1, 1, dB)
        return carry
    lax.fori_loop(0, NCH // 2, cpair_body, 0)
    compute_codes(NCH - 1, 0, load_cols(NCH - 1, 0))


@functools.partial(
    pl.kernel,
    out_type=jax.ShapeDtypeStruct((NC, NP, H), jnp.float32),
    mesh=_mesh,
    scratch_types=[
        pltpu.VMEM_SHARED((NB, H), jnp.float32),
        pltpu.VMEM_SHARED((NP, H), jnp.float32),
        [pltpu.VMEM((EC,), jnp.int32)] * 2,
        [pltpu.VMEM((EC,), jnp.int32)] * 2,
        [pltpu.VMEM((EC,), jnp.int32)] * 2,
        [pltpu.VMEM((EC, H), jnp.float32)] * 2,
        [pltpu.VMEM((EC, H), jnp.float32)] * 2,
        [[pltpu.SemaphoreType.DMA] * 2] * 2,
        pltpu.SemaphoreType.DMA,
        pltpu.SemaphoreType.DMA,
        pltpu.SemaphoreType.DMA,
    ],
)
def _layer_sc_kernel(x, src, dst, code, etab,
                     agg_out,
                     etab_sh, agg_sh, sr, dx, co, xr, er, sg, ssc, sia, sib):
    c = lax.axis_index("c")
    s = lax.axis_index("s")
    wid = s * NC + c

    @pl.when(s == 0)
    def _():
        pltpu.sync_copy(etab, etab_sh)

    _zero_buf(xr[0], EC)
    node_base = s * NODE_PER
    off = 0
    while off < NODE_PER:
        size = min(EC, NODE_PER - off)
        pltpu.sync_copy(xr[0].at[pl.ds(0, size)],
                        agg_sh.at[pl.ds(node_base + off, size)])
        off += size
    plsc.subcore_barrier()

    def load_idx_async(i, q, sem):
        sl = pl.ds(wid * E_PER + i * EC, EC)
        return [pltpu.async_copy(src.at[sl], sr[q], sem),
                pltpu.async_copy(code.at[sl], co[q], sem),
                pltpu.async_copy(dst.at[sl], dx[q], sem)]

    def pair_body(k, carry):
        idxA = load_idx_async(2 * k, 0, sia)
        idxB = load_idx_async(2 * k + 1, 1, sib)
        for d in idxA:
            d.wait()
        gxA = pltpu.async_copy(x.at[sr[0]], xr[0], sg[0][0])
        geA = pltpu.async_copy(etab_sh.at[co[0]], er[0], sg[0][1])
        for d in idxB:
            d.wait()
        gxB = pltpu.async_copy(x.at[sr[1]], xr[1], sg[1][0])
        gxA.wait()
        geA.wait()
        geB = pltpu.async_copy(etab_sh.at[co[1]], er[1], sg[1][1])
        _relu_add_rows(xr[0], er[0])
        gxB.wait()
        geB.wait()
        scA = pltpu.async_copy(xr[0], agg_sh.at[dx[0]], ssc, add=True)
        _relu_add_rows(xr[1], er[1])
        scA.wait()
        pltpu.sync_copy(xr[1], agg_sh.at[dx[1]], add=True)
        return carry
    lax.fori_loop(0, NCH // 2, pair_body, 0)

    for d in load_idx_async(NCH - 1, 0, sia):
        d.wait()
    gxA = pltpu.async_copy(x.at[sr[0]], xr[0], sg[0][0])
    geA = pltpu.async_copy(etab_sh.at[co[0]], er[0], sg[0][1])
    gxA.wait()
    geA.wait()
    _relu_add_rows(xr[0], er[0])
    pltpu.sync_copy(xr[0], agg_sh.at[dx[0]], add=True)
    plsc.subcore_barrier()

    off = 0
    while off < NODE_PER:
        size = min(EC, NODE_PER - off)
        pltpu.sync_copy(agg_sh.at[pl.ds(node_base + off, size)],
                        agg_out.at[c, pl.ds(node_base + off, size)])
        off += size


ROWS_BLK = NODE_PER
N_BLKS = NP // ROWS_BLK


def _tc_layer_body(x_ref, agg_ref, w_ref, b_ref, xn_ref, pooled_ref,
                   pooledx_ref):
    i = pl.program_id(0)
    xb = x_ref[...]
    acc = xb + agg_ref[0] + agg_ref[1]
    y = jnp.dot(acc, w_ref[...], preferred_element_type=jnp.float32)
    y = jnp.maximum(y + b_ref[...], 0.0)
    rows = i * ROWS_BLK + lax.broadcasted_iota(jnp.int32, (ROWS_BLK, 1), 0)
    valid = rows < N
    y = jnp.where(valid, y, 0.0)
    xn_ref[...] = y
    ps = jnp.sum(y, axis=0, keepdims=True)

    @pl.when(i == 0)
    def _():
        pooled_ref[...] = ps

    @pl.when(i > 0)
    def _():
        pooled_ref[...] += ps

    if pooledx_ref is not None:
        pxs = jnp.sum(jnp.where(valid, xb, 0.0), axis=0, keepdims=True)

        @pl.when(i == 0)
        def _():
            pooledx_ref[...] = pxs

        @pl.when(i > 0)
        def _():
            pooledx_ref[...] += pxs


def _make_tc_layer(with_x_pool):
    out_shapes = [
        jax.ShapeDtypeStruct((NP, H), jnp.float32),
        jax.ShapeDtypeStruct((1, H), jnp.float32),
    ]
    out_specs = [
        pl.BlockSpec((ROWS_BLK, H), lambda i: (i, 0)),
        pl.BlockSpec((1, H), lambda i: (0, 0)),
    ]
    if with_x_pool:
        out_shapes.append(jax.ShapeDtypeStruct((1, H), jnp.float32))
        out_specs.append(pl.BlockSpec((1, H), lambda i: (0, 0)))
        body = _tc_layer_body
    else:
        def body(x_ref, agg_ref, w_ref, b_ref, xn_ref, pooled_ref):
            _tc_layer_body(x_ref, agg_ref, w_ref, b_ref, xn_ref, pooled_ref,
                           None)
    return pl.pallas_call(
        body,
        grid=(N_BLKS,),
        in_specs=[
            pl.BlockSpec((ROWS_BLK, H), lambda i: (i, 0)),
            pl.BlockSpec((NC, ROWS_BLK, H), lambda i: (0, i, 0)),
            pl.BlockSpec((H, H), lambda i: (0, 0)),
            pl.BlockSpec((1, H), lambda i: (0, 0)),
        ],
        out_specs=out_specs,
        out_shape=out_shapes,
    )


_tc_layer_first = _make_tc_layer(True)
_tc_layer_rest = _make_tc_layer(False)


def _head_body(pooled_ref, linw_ref, linb_ref, out_ref):
    s = jnp.sum(pooled_ref[...] * linw_ref[...]) + linb_ref[0, 0]
    out_ref[...] = s.reshape(1, 1)


_head = pl.pallas_call(
    _head_body,
    out_shape=jax.ShapeDtypeStruct((1, 1), jnp.float32),
)


@jax.jit
def kernel(atom_table, bond0, bond1, bond2, W, b, lin_W, lin_b,
           atom_idx, edge_index, edge_attr):
    atom_idx = atom_idx.astype(jnp.int32)
    edge_index = edge_index.astype(jnp.int32)
    edge_attr = edge_attr.astype(jnp.int32)
    src = edge_index[0]
    dst = edge_index[1]
    ea0 = edge_attr[:, 0]
    ea1 = edge_attr[:, 1]
    ea2 = edge_attr[:, 2]

    x0, etab, code = _encode_kernel(atom_table, atom_idx, bond0, bond1,
                                    bond2, ea0, ea1, ea2)

    pooled = []
    x = x0
    for l in range(NLAYER):
        agg2 = _layer_sc_kernel(x, src, dst, code, etab)
        wl = W[l]
        bl = b[l].reshape(1, H)
        if l == 0:
            x, p, p0 = _tc_layer_first(x, agg2, wl, bl)
            pooled.append(p0)
        else:
            x, p = _tc_layer_rest(x, agg2, wl, bl)
        pooled.append(p)

    pooled_all = jnp.concatenate(pooled, axis=0)
    pooled_all = jnp.pad(pooled_all, ((0, 2), (0, 0)))
    linw = jnp.pad(lin_W.reshape(NLAYER + 1, H), ((0, 2), (0, 0)))
    linb = lin_b.reshape(1, 1)
    out = _head(pooled_all, linw, linb)
    return out.reshape(1)
